# baseline (speedup 1.0000x reference)
import jax
import jax.numpy as jnp
from jax import lax
from jax.experimental import pallas as pl
from jax.experimental.pallas import tpu as pltpu, tpu_sc as plsc

N = 500000
DIM = 128
EPAD = 128
CH = 256
LANES = 16
NW = 32

NUM_FULL = N // CH
T_COMMON = NUM_FULL // NW
TAIL = N - NUM_FULL * CH
TAIL_BASE = NUM_FULL * CH


def _search16(edges_v, v):
    pos = jnp.zeros((LANES,), jnp.int32)
    step = EPAD // 2
    while step >= 1:
        probe = pos + (step - 1)
        ev = plsc.load_gather(edges_v, [probe])
        pos = jnp.where(ev < v, pos + step, pos)
        step //= 2
    return pos


def _body(time_hbm, edges_hbm, table_hbm, out_hbm,
          edges_v, table_v, tv0, tv1, idx0, idx1, rows0, rows1,
          tvt, idxt, rowst,
          tvs0, tvs1, gs0, gs1, ws0, ws1, tsem):
    nc = lax.axis_index("c")
    ns = lax.axis_index("s")
    wid = ns * 2 + nc

    tv = (tv0, tv1)
    idx = (idx0, idx1)
    rows = (rows0, rows1)
    tvs = (tvs0, tvs1)
    gs = (gs0, gs1)
    ws = (ws0, ws1)

    pltpu.sync_copy(edges_hbm, edges_v)
    pltpu.sync_copy(table_hbm, table_v)

    def tv_slice(t):
        return time_hbm.at[pl.ds((wid + t * NW) * CH, CH)]

    def out_slice(t):
        return out_hbm.at[pl.ds((wid + t * NW) * CH, CH)]

    def start_tv(t, b):
        pltpu.async_copy(tv_slice(t), tv[b], tvs[b])

    def wait_tv(t, b):
        pltpu.make_async_copy(tv_slice(t), tv[b], tvs[b]).wait()

    def compute_idx(b):
        tvb, idxb = tv[b], idx[b]
        for j in range(CH // 128):
            row = idxb.at[j]
            for k in range(128 // LANES):
                v = tvb[pl.ds(j * 128 + k * LANES, LANES)]
                row[pl.ds(k * LANES, LANES)] = _search16(edges_v, v)

    def start_gathers(b):
        for j in range(CH // 128):
            pltpu.async_copy(table_v.at[idx[b].at[j]],
                             rows[b].at[pl.ds(j * 128, 128)], gs[b])

    def wait_gathers(b):
        for j in range(CH // 128):
            pltpu.make_async_copy(table_v.at[idx[b].at[j]],
                                  rows[b].at[pl.ds(j * 128, 128)], gs[b]).wait()

    def start_write(t, b):
        pltpu.async_copy(rows[b], out_slice(t), ws[b])

    def wait_write(t, b):
        pltpu.make_async_copy(rows[b], out_slice(t), ws[b]).wait()

    start_tv(0, 0)
    start_tv(1, 1)

    def pair(i, carry):
        for tt in range(2):
            b = tt
            o = 1 - tt
            t = 2 * i + tt

            def write_prev():
                wait_gathers(o)
                start_write(t - 1, o)
            if tt == 0:
                pl.when(i >= 1)(write_prev)
            else:
                write_prev()

            wait_tv(t, b)
            compute_idx(b)

            pl.when(i >= 1)(lambda: wait_write(t - 2, b))

            start_gathers(b)

            if tt == 0:
                start_tv(t + 2, b)
            else:
                pl.when(jnp.logical_or(i < T_COMMON // 2 - 1, wid == 0))(
                    lambda: start_tv(t + 2, b))
        return carry

    lax.fori_loop(0, T_COMMON // 2, pair, 0)

    t = T_COMMON - 1
    wait_gathers(1)
    start_write(t - 1, 1)
    wait_tv(t, 0)
    compute_idx(0)
    wait_write(t - 2, 0)
    start_gathers(0)
    wait_gathers(0)
    start_write(t, 0)
    wait_write(t - 1, 1)
    wait_write(t, 0)

    @pl.when(wid == 0)
    def _extra():
        te = T_COMMON
        wait_tv(te, 1)
        compute_idx(1)
        start_gathers(1)
        wait_gathers(1)
        start_write(te, 1)
        wait_write(te, 1)

    @pl.when(wid == NW - 1)
    def _tail():
        pltpu.sync_copy(time_hbm.at[pl.ds(TAIL_BASE, TAIL)], tvt)
        for k in range(TAIL // LANES):
            v = tvt[pl.ds(k * LANES, LANES)]
            idxt[pl.ds(k * LANES, LANES)] = _search16(edges_v, v)
        pltpu.async_copy(table_v.at[idxt], rowst, tsem).wait()
        pltpu.sync_copy(rowst, out_hbm.at[pl.ds(TAIL_BASE, TAIL)])


@jax.jit
def _run(time_value, edges_pad, table):
    mesh = plsc.VectorSubcoreMesh(core_axis_name="c", subcore_axis_name="s")
    return pl.kernel(
        _body,
        out_type=jax.ShapeDtypeStruct((N, DIM), jnp.float32),
        mesh=mesh,
        compiler_params=pltpu.CompilerParams(needs_layout_passes=False),
        scratch_types=[
            pltpu.VMEM((EPAD,), jnp.float32),
            pltpu.VMEM_SHARED((101, DIM), jnp.float32),
            pltpu.VMEM((CH,), jnp.float32),
            pltpu.VMEM((CH,), jnp.float32),
            pltpu.VMEM((CH // 128, 128), jnp.int32),
            pltpu.VMEM((CH // 128, 128), jnp.int32),
            pltpu.VMEM((CH, DIM), jnp.float32),
            pltpu.VMEM((CH, DIM), jnp.float32),
            pltpu.VMEM((TAIL,), jnp.float32),
            pltpu.VMEM((TAIL,), jnp.int32),
            pltpu.VMEM((TAIL, DIM), jnp.float32),
            pltpu.SemaphoreType.DMA,
            pltpu.SemaphoreType.DMA,
            pltpu.SemaphoreType.DMA,
            pltpu.SemaphoreType.DMA,
            pltpu.SemaphoreType.DMA,
            pltpu.SemaphoreType.DMA,
            pltpu.SemaphoreType.DMA,
        ],
    )(time_value, edges_pad, table)


def kernel(time_value, absolute_bin_edges, ab_duration_embed):
    edges_pad = jnp.concatenate(
        [absolute_bin_edges.astype(jnp.float32),
         jnp.full((EPAD - absolute_bin_edges.shape[0],), jnp.inf, jnp.float32)]
    )
    return _run(time_value, edges_pad, ab_duration_embed)


# speedup vs baseline: 1.0211x; 1.0211x over previous
import jax
import jax.numpy as jnp
from jax import lax
from jax.experimental import pallas as pl
from jax.experimental.pallas import tpu as pltpu, tpu_sc as plsc

N = 500000
DIM = 128
EPAD = 128
CH = 384
LANES = 16
NW = 32

NUM_FULL = N // CH
T_COMMON = NUM_FULL // NW
N_EXTRA = NUM_FULL - T_COMMON * NW
TAIL = N - NUM_FULL * CH
TAIL_BASE = NUM_FULL * CH


def _search16(edges_v, v):
    pos = jnp.zeros((LANES,), jnp.int32)
    step = EPAD // 2
    while step >= 1:
        probe = pos + (step - 1)
        ev = plsc.load_gather(edges_v, [probe])
        pos = jnp.where(ev < v, pos + step, pos)
        step //= 2
    return pos


def _body(time_hbm, edges_hbm, table_hbm, out_hbm,
          edges_v, table_v, tv0, tv1, idx0, idx1, rows0, rows1,
          tvt, idxt, rowst,
          tvs0, tvs1, gs0, gs1, ws0, ws1, tsem):
    nc = lax.axis_index("c")
    ns = lax.axis_index("s")
    wid = ns * 2 + nc

    tv = (tv0, tv1)
    idx = (idx0, idx1)
    rows = (rows0, rows1)
    tvs = (tvs0, tvs1)
    gs = (gs0, gs1)
    ws = (ws0, ws1)

    pltpu.sync_copy(edges_hbm, edges_v)
    pltpu.sync_copy(table_hbm, table_v)

    def tv_slice(t):
        return time_hbm.at[pl.ds((wid + t * NW) * CH, CH)]

    def out_slice(t):
        return out_hbm.at[pl.ds((wid + t * NW) * CH, CH)]

    def start_tv(t, b):
        pltpu.async_copy(tv_slice(t), tv[b], tvs[b])

    def wait_tv(t, b):
        pltpu.make_async_copy(tv_slice(t), tv[b], tvs[b]).wait()

    def compute_idx(b):
        tvb, idxb = tv[b], idx[b]
        for j in range(CH // 128):
            row = idxb.at[j]

            def kb(k, c, j=j, row=row, tvb=tvb):
                v = tvb[pl.ds(j * 128 + k * LANES, LANES)]
                row[pl.ds(k * LANES, LANES)] = _search16(edges_v, v)
                return c

            lax.fori_loop(0, 128 // LANES, kb, 0)

    def start_gathers(b):
        for j in range(CH // 128):
            pltpu.async_copy(table_v.at[idx[b].at[j]],
                             rows[b].at[pl.ds(j * 128, 128)], gs[b])

    def wait_gathers(b):
        for j in range(CH // 128):
            pltpu.make_async_copy(table_v.at[idx[b].at[j]],
                                  rows[b].at[pl.ds(j * 128, 128)], gs[b]).wait()

    def start_write(t, b):
        pltpu.async_copy(rows[b], out_slice(t), ws[b])

    def wait_write(t, b):
        pltpu.make_async_copy(rows[b], out_slice(t), ws[b]).wait()

    start_tv(0, 0)
    start_tv(1, 1)

    def pair(i, carry):
        for tt in range(2):
            b = tt
            o = 1 - tt
            t = 2 * i + tt

            def write_prev():
                wait_gathers(o)
                start_write(t - 1, o)
            if tt == 0:
                pl.when(i >= 1)(write_prev)
            else:
                write_prev()

            wait_tv(t, b)
            compute_idx(b)

            pl.when(i >= 1)(lambda: wait_write(t - 2, b))

            start_gathers(b)

            if tt == 0:
                pl.when(jnp.logical_or(i < T_COMMON // 2 - 1, wid < N_EXTRA))(
                    lambda: start_tv(t + 2, b))
            else:
                pl.when(i < T_COMMON // 2 - 1)(lambda: start_tv(t + 2, b))
        return carry

    lax.fori_loop(0, T_COMMON // 2, pair, 0)

    t = T_COMMON - 1
    wait_gathers(1)
    start_write(t, 1)
    wait_write(t - 1, 0)
    wait_write(t, 1)

    @pl.when(wid < N_EXTRA)
    def _extra():
        te = T_COMMON
        wait_tv(te, 0)
        compute_idx(0)
        start_gathers(0)
        wait_gathers(0)
        start_write(te, 0)
        wait_write(te, 0)

    @pl.when(wid == NW - 1)
    def _tail():
        pltpu.sync_copy(time_hbm.at[pl.ds(TAIL_BASE, TAIL)], tvt)
        for k in range(TAIL // LANES):
            v = tvt[pl.ds(k * LANES, LANES)]
            idxt[pl.ds(k * LANES, LANES)] = _search16(edges_v, v)
        pltpu.async_copy(table_v.at[idxt], rowst, tsem).wait()
        pltpu.sync_copy(rowst, out_hbm.at[pl.ds(TAIL_BASE, TAIL)])


@jax.jit
def _run(time_value, edges_pad, table):
    mesh = plsc.VectorSubcoreMesh(core_axis_name="c", subcore_axis_name="s")
    return pl.kernel(
        _body,
        out_type=jax.ShapeDtypeStruct((N, DIM), jnp.float32),
        mesh=mesh,
        compiler_params=pltpu.CompilerParams(needs_layout_passes=False),
        scratch_types=[
            pltpu.VMEM((EPAD,), jnp.float32),
            pltpu.VMEM_SHARED((101, DIM), jnp.float32),
            pltpu.VMEM((CH,), jnp.float32),
            pltpu.VMEM((CH,), jnp.float32),
            pltpu.VMEM((CH // 128, 128), jnp.int32),
            pltpu.VMEM((CH // 128, 128), jnp.int32),
            pltpu.VMEM((CH, DIM), jnp.float32),
            pltpu.VMEM((CH, DIM), jnp.float32),
            pltpu.VMEM((TAIL,), jnp.float32),
            pltpu.VMEM((TAIL,), jnp.int32),
            pltpu.VMEM((TAIL, DIM), jnp.float32),
            pltpu.SemaphoreType.DMA,
            pltpu.SemaphoreType.DMA,
            pltpu.SemaphoreType.DMA,
            pltpu.SemaphoreType.DMA,
            pltpu.SemaphoreType.DMA,
            pltpu.SemaphoreType.DMA,
            pltpu.SemaphoreType.DMA,
        ],
    )(time_value, edges_pad, table)


def kernel(time_value, absolute_bin_edges, ab_duration_embed):
    edges_pad = jnp.concatenate(
        [absolute_bin_edges.astype(jnp.float32),
         jnp.full((EPAD - absolute_bin_edges.shape[0],), jnp.inf, jnp.float32)]
    )
    return _run(time_value, edges_pad, ab_duration_embed)
